# baseline (speedup 1.0000x reference)
import functools

import jax
import jax.numpy as jnp
from jax import lax
from jax.experimental import pallas as pl
from jax.experimental.pallas import tpu as pltpu
from jax.experimental.pallas import tpu_sc as plsc


def _probe():
    mesh = plsc.VectorSubcoreMesh(core_axis_name="c", subcore_axis_name="s")

    @functools.partial(
        pl.kernel,
        out_type=jax.ShapeDtypeStruct((16,), jnp.float32),
        mesh=mesh,
        compiler_params=pltpu.CompilerParams(needs_layout_passes=False),
        scratch_types=[
            pltpu.VMEM((64,), jnp.float32),
            pltpu.VMEM((16, 16), jnp.float32),
            pltpu.VMEM((16,), jnp.int32),
            pltpu.VMEM((16,), jnp.float32),
            pltpu.VMEM_SHARED((16, 16), jnp.float32),
        ],
    )
    def probe(out_hbm, acc_v, rows_v, idx_v, res_v, sh):
        cid = lax.axis_index("c")
        sid = lax.axis_index("s")
        t00 = jnp.logical_and(cid == 0, sid == 0)

        @pl.when(t00)
        def _():
            zeros = jnp.zeros((16,), jnp.float32)
            ones = jnp.ones((16,), jnp.float32)
            iota = lax.iota(jnp.int32, 16)

            for j in range(4):
                acc_v[pl.ds(j * 16, 16)] = zeros
            k = iota % 4
            plsc.addupdate_scatter(acc_v, [k], ones)
            got = acc_v[pl.ds(0, 16)]
            exp = jnp.where(iota < 4, 4.0, 0.0)
            err_a = jnp.sum(jnp.abs(got - exp))

            for r in range(16):
                rows_v[r, :] = (iota + 16 * r).astype(jnp.float32)
            pltpu.sync_copy(rows_v, sh)
            res = jnp.where(iota == 0, err_a, 0.0)
            res_v[...] = res

        plsc.subcore_barrier()

        @pl.when(t00)
        def _():
            zeros = jnp.zeros((16,), jnp.float32)
            ones = jnp.ones((16,), jnp.float32)
            iota = lax.iota(jnp.int32, 16)

            for r in range(16):
                rows_v[r, :] = zeros
            pltpu.sync_copy(sh, rows_v)
            accum = zeros
            for r in range(16):
                accum = accum + jnp.abs(rows_v[r, :] - (iota + 16 * r).astype(jnp.float32))
            b0 = jnp.sum(accum)
            res_v[...] = res_v[...] + jnp.where(iota == 3, b0, 0.0)

            for r in range(16):
                rows_v[r, :] = zeros
            pltpu.sync_copy(rows_v, sh)
            for r in range(16):
                rows_v[r, :] = ones
            idx_v[...] = iota % 2
            pltpu.sync_copy(rows_v, sh.at[idx_v], add=True)

        plsc.subcore_barrier()

        @pl.when(t00)
        def _():
            zeros = jnp.zeros((16,), jnp.float32)
            iota = lax.iota(jnp.int32, 16)
            b0v = res_v[...]
            b0 = jnp.sum(jnp.where(iota == 3, b0v, 0.0))
            for r in range(16):
                rows_v[r, :] = zeros
            pltpu.sync_copy(sh, rows_v)
            accum = zeros
            for r in range(16):
                exp_r = jnp.full((16,), 8.0, jnp.float32) if r < 2 else zeros
                accum = accum + jnp.abs(rows_v[r, :] - exp_r)
            b1 = jnp.sum(accum)
            err_b = (
                jnp.where(b0 > 0.001, 1.0, 0.0)
                + jnp.where(b1 > 0.001, 2.0, 0.0)
                + jnp.minimum(b1, 500.0) * 0.001
            )
            res_v[...] = b0v + jnp.where(iota == 1, err_b, 0.0)
            pltpu.sync_copy(res_v, out_hbm)

    return probe()


def _conv1d(h, w, b):
    y = lax.conv_general_dilated(
        h, w, window_strides=(1,), padding=((1, 1),),
        dimension_numbers=("NCH", "OIH", "NCH"),
    )
    return y + b[None, :, None]


def _sage(h, src, dst, Wl, bl, Wr, num_nodes):
    msgs = jnp.take(h, src, axis=0)
    agg = jax.ops.segment_sum(msgs, dst, num_segments=num_nodes)
    deg = jax.ops.segment_sum(jnp.ones((src.shape[0],), h.dtype), dst, num_segments=num_nodes)
    agg = agg / jnp.maximum(deg, 1.0)[:, None]
    return agg @ Wl.T + bl + h @ Wr.T


def kernel(x, edge_index, W1, b1, W2, b2, Wl1, bl1, Wr1, Wl2, bl2, Wr2, Wh, bh):
    src, dst = edge_index[0], edge_index[1]
    h = x[:, None, :]
    h = jax.nn.relu(_conv1d(h, W1, b1))
    h = jax.nn.relu(_conv1d(h, W2, b2))
    h = jnp.mean(h, axis=2)
    h = jax.nn.relu(_sage(h, src, dst, Wl1, bl1, Wr1, x.shape[0]))
    h = _sage(h, src, dst, Wl2, bl2, Wr2, x.shape[0])
    out = (h @ Wh.T + bh).squeeze(-1)
    p = _probe()
    return out + p[_PROBE_SELECT]


_PROBE_SELECT = 1


# speedup vs baseline: 8.9884x; 8.9884x over previous
import functools

import jax
import jax.numpy as jnp
from jax import lax
from jax.experimental import pallas as pl
from jax.experimental.pallas import tpu as pltpu
from jax.experimental.pallas import tpu_sc as plsc

N = 50000
E = 1600000
NT = 50176
EP = 1605632
EPW = EP // 32
NCHUNK = EPW // 1024
BLK = 1024
RPT = NT // 16

_SC_PARAMS = pltpu.CompilerParams(
    needs_layout_passes=False, use_tc_tiling_on_sc=False
)


def _tc1_body(xb, w1, b1r, w2f, b2r, ob):
    xx = xb[...]
    z1 = jnp.zeros((BLK, 1), jnp.float32)
    xm = jnp.concatenate([z1, xx[:, :-1]], axis=1)
    xp = jnp.concatenate([xx[:, 1:], z1], axis=1)
    xs = jnp.stack([xm, xx, xp], axis=1)
    h1 = jnp.einsum("bkl,ck->bcl", xs, w1[...],
                    preferred_element_type=jnp.float32)
    h1 = jnp.maximum(h1 + b1r[...][None, :, None], 0.0)
    z2 = jnp.zeros((BLK, 16, 1), jnp.float32)
    h1m = jnp.concatenate([z2, h1[:, :, :-1]], axis=2)
    h1p = jnp.concatenate([h1[:, :, 1:], z2], axis=2)
    hh = jnp.concatenate([h1m, h1, h1p], axis=1)
    h2 = jnp.einsum("bil,oi->bol", hh, w2f[...],
                    preferred_element_type=jnp.float32)
    h2 = jnp.maximum(h2 + b2r[...][None, :, None], 0.0)
    ob[...] = jnp.mean(h2, axis=2)


def _tc1(xpad, w1, b1, w2f, b2):
    return pl.pallas_call(
        _tc1_body,
        grid=(NT // BLK,),
        in_specs=[
            pl.BlockSpec((BLK, 32), lambda i: (i, 0)),
            pl.BlockSpec((16, 3), lambda i: (0, 0)),
            pl.BlockSpec((16,), lambda i: (0,)),
            pl.BlockSpec((32, 48), lambda i: (0, 0)),
            pl.BlockSpec((32,), lambda i: (0,)),
        ],
        out_specs=pl.BlockSpec((BLK, 32), lambda i: (i, 0)),
        out_shape=jax.ShapeDtypeStruct((NT, 32), jnp.float32),
    )(xpad, w1, b1, w2f, b2)


def _sca_body(h0p, srcp, dstp, acc1, rows_v, sidx, didx, sem, acc):
    cid = lax.axis_index("c")
    sid = lax.axis_index("s")
    wid = cid * 16 + sid
    zeros = jnp.zeros((16,), jnp.float32)

    def zrows(i, carry):
        rows_v[i, pl.ds(0, 16)] = zeros
        rows_v[i, pl.ds(16, 16)] = zeros
        return carry

    lax.fori_loop(0, 512, zrows, 0)

    base = sid * RPT
    for q in range(6):
        pltpu.sync_copy(rows_v, acc.at[pl.ds(base + q * 512, 512)])
    pltpu.sync_copy(rows_v.at[pl.ds(0, RPT - 3072)],
                    acc.at[pl.ds(base + 3072, RPT - 3072)])
    plsc.subcore_barrier()

    def chunk(g, carry):
        e0 = wid * EPW + g * 512
        pltpu.sync_copy(srcp.at[pl.ds(e0, 512)], sidx)
        pltpu.sync_copy(dstp.at[pl.ds(e0, 512)], didx)
        pltpu.async_copy(h0p.at[sidx], rows_v, sem).wait()
        pltpu.sync_copy(rows_v, acc.at[didx], add=True)
        return carry

    lax.fori_loop(0, EPW // 512, chunk, 0)
    plsc.subcore_barrier()

    pltpu.sync_copy(acc.at[pl.ds(base, RPT)], acc1.at[cid, pl.ds(base, RPT)])


def _sca(h0p, srcp, dstp):
    mesh = plsc.VectorSubcoreMesh(core_axis_name="c", subcore_axis_name="s")
    return pl.kernel(
        _sca_body,
        out_type=pltpu.MemorySpace.HBM((2, NT, 32), jnp.float32),
        mesh=mesh,
        compiler_params=_SC_PARAMS,
        scratch_types=[
            pltpu.VMEM((512, 32), jnp.float32),
            pltpu.VMEM((512,), jnp.int32),
            pltpu.VMEM((512,), jnp.int32),
            pltpu.SemaphoreType.DMA,
            pltpu.VMEM_SHARED((NT, 32), jnp.float32),
        ],
    )(h0p, srcp, dstp)


def _scd_body(dstp, degp, hist, didx):
    cid = lax.axis_index("c")
    sid = lax.axis_index("s")
    wid = cid * 16 + sid
    zeros = jnp.zeros((16,), jnp.float32)
    ones = jnp.ones((16,), jnp.float32)

    def zhist(i, carry):
        hist[pl.ds(i * 16, 16)] = zeros
        return carry

    lax.fori_loop(0, NT // 16, zhist, 0)

    def chunk(g, carry):
        e0 = wid * EPW + g * 1024
        pltpu.sync_copy(dstp.at[pl.ds(e0, 1024)], didx)

        def hinc(i, c2):
            dvec = didx[pl.ds(i * 16, 16)]
            plsc.addupdate_scatter(hist, [dvec], ones)
            return c2

        lax.fori_loop(0, 64, hinc, 0)
        return carry

    lax.fori_loop(0, NCHUNK, chunk, 0)
    pltpu.sync_copy(hist, degp.at[wid])


def _scd(dstp):
    mesh = plsc.VectorSubcoreMesh(core_axis_name="c", subcore_axis_name="s")
    return pl.kernel(
        _scd_body,
        out_type=pltpu.MemorySpace.HBM((32, NT), jnp.float32),
        mesh=mesh,
        compiler_params=_SC_PARAMS,
        scratch_types=[
            pltpu.VMEM((NT,), jnp.float32),
            pltpu.VMEM((1024,), jnp.int32),
        ],
    )(dstp)


def _tc2_body(a1b, dgb, h0b, wl1, bl1r, wr1, wlv, wrv, cb, sb, tb, dib):
    a = a1b[0] + a1b[1]
    deg = jnp.sum(dgb[...], axis=0)
    degc = jnp.maximum(deg, 1.0)
    agg = a / degc[:, None]
    h1 = (jnp.einsum("bf,hf->bh", agg, wl1[...],
                     preferred_element_type=jnp.float32)
          + bl1r[...][None, :]
          + jnp.einsum("bf,hf->bh", h0b[...], wr1[...],
                       preferred_element_type=jnp.float32))
    h1 = jnp.maximum(h1, 0.0)
    sb[...] = jnp.sum(h1 * wlv[...][None, :], axis=1)
    tb[...] = jnp.sum(h1 * wrv[...][None, :], axis=1) + cb[0]
    dib[...] = 1.0 / degc


def _tc2(acc1, degp, h0p, Wl1, bl1, Wr1, wl, wr, cc):
    return pl.pallas_call(
        _tc2_body,
        grid=(NT // BLK,),
        in_specs=[
            pl.BlockSpec((2, BLK, 32), lambda i: (0, i, 0)),
            pl.BlockSpec((32, BLK), lambda i: (0, i)),
            pl.BlockSpec((BLK, 32), lambda i: (i, 0)),
            pl.BlockSpec((64, 32), lambda i: (0, 0)),
            pl.BlockSpec((64,), lambda i: (0,)),
            pl.BlockSpec((64, 32), lambda i: (0, 0)),
            pl.BlockSpec((64,), lambda i: (0,)),
            pl.BlockSpec((64,), lambda i: (0,)),
            pl.BlockSpec(memory_space=pltpu.SMEM),
        ],
        out_specs=[
            pl.BlockSpec((BLK,), lambda i: (i,)),
            pl.BlockSpec((BLK,), lambda i: (i,)),
            pl.BlockSpec((BLK,), lambda i: (i,)),
        ],
        out_shape=[
            jax.ShapeDtypeStruct((NT,), jnp.float32),
            jax.ShapeDtypeStruct((NT,), jnp.float32),
            jax.ShapeDtypeStruct((NT,), jnp.float32),
        ],
    )(acc1, degp, h0p, Wl1, bl1, Wr1, wl, wr, cc)


def _scb_body(sp, srcp, dstp, ssump, sv, accv, sidx, didx):
    cid = lax.axis_index("c")
    sid = lax.axis_index("s")
    wid = cid * 16 + sid
    zeros = jnp.zeros((16,), jnp.float32)

    pltpu.sync_copy(sp, sv)

    def zacc(i, carry):
        accv[pl.ds(i * 16, 16)] = zeros
        return carry

    lax.fori_loop(0, NT // 16, zacc, 0)

    def chunk(g, carry):
        e0 = wid * EPW + g * 1024
        pltpu.sync_copy(srcp.at[pl.ds(e0, 1024)], sidx)
        pltpu.sync_copy(dstp.at[pl.ds(e0, 1024)], didx)

        def grp(i, c2):
            svec = plsc.load_gather(sv, [sidx[pl.ds(i * 16, 16)]])
            plsc.addupdate_scatter(accv, [didx[pl.ds(i * 16, 16)]], svec)
            return c2

        lax.fori_loop(0, 64, grp, 0)
        return carry

    lax.fori_loop(0, NCHUNK, chunk, 0)
    pltpu.sync_copy(accv, ssump.at[wid])


def _scb(sp, srcp, dstp):
    mesh = plsc.VectorSubcoreMesh(core_axis_name="c", subcore_axis_name="s")
    return pl.kernel(
        _scb_body,
        out_type=pltpu.MemorySpace.HBM((32, NT), jnp.float32),
        mesh=mesh,
        compiler_params=_SC_PARAMS,
        scratch_types=[
            pltpu.VMEM((NT,), jnp.float32),
            pltpu.VMEM((NT,), jnp.float32),
            pltpu.VMEM((1024,), jnp.int32),
            pltpu.VMEM((1024,), jnp.int32),
        ],
    )(sp, srcp, dstp)


def _tc3_body(ssb, dib, tb, ob):
    ob[...] = jnp.sum(ssb[...], axis=0) * dib[...] + tb[...]


def _tc3(ssump, dinv, t):
    return pl.pallas_call(
        _tc3_body,
        grid=(NT // BLK,),
        in_specs=[
            pl.BlockSpec((32, BLK), lambda i: (0, i)),
            pl.BlockSpec((BLK,), lambda i: (i,)),
            pl.BlockSpec((BLK,), lambda i: (i,)),
        ],
        out_specs=pl.BlockSpec((BLK,), lambda i: (i,)),
        out_shape=jax.ShapeDtypeStruct((NT,), jnp.float32),
    )(ssump, dinv, t)


def kernel(x, edge_index, W1, b1, W2, b2, Wl1, bl1, Wr1, Wl2, bl2, Wr2, Wh, bh):
    src = edge_index[0]
    dst = edge_index[1]
    padv = jnp.full((EP - E,), N, jnp.int32)
    srcp = jnp.concatenate([src, padv])
    dstp = jnp.concatenate([dst, padv])
    xpad = jnp.pad(x, ((0, NT - N), (0, 0)))

    w1 = W1[:, 0, :]
    w2f = jnp.concatenate([W2[:, :, 0], W2[:, :, 1], W2[:, :, 2]], axis=1)
    wl = (Wh @ Wl2)[0]
    wr = (Wh @ Wr2)[0]
    cc = Wh @ bl2 + bh

    h0p = _tc1(xpad, w1, b1, w2f, b2)
    degp = _scd(dstp)
    acc1 = _sca(h0p, srcp, dstp)
    s, t, dinv = _tc2(acc1, degp, h0p, Wl1, bl1, Wr1, wl, wr, cc)
    ssump = _scb(s, srcp, dstp)
    outp = _tc3(ssump, dinv, t)
    return outp[:N]


# speedup vs baseline: 16.0443x; 1.7850x over previous
import functools

import jax
import jax.numpy as jnp
from jax import lax
from jax.experimental import pallas as pl
from jax.experimental.pallas import tpu as pltpu
from jax.experimental.pallas import tpu_sc as plsc

N = 50000
E = 1600000
NT = 50176
EP = 1605632
EPW = EP // 32
NCHUNK = EPW // 1024
BLK = 1024
RPT = NT // 16

_SC_PARAMS = pltpu.CompilerParams(
    needs_layout_passes=False, use_tc_tiling_on_sc=False
)


def _tc1_body(xb, t1, b1b, t2, b2b, ob):
    h1 = jnp.maximum(
        jnp.dot(xb[...], t1[...], preferred_element_type=jnp.float32)
        + b1b[...][None, :], 0.0)
    h2 = jnp.maximum(
        jnp.dot(h1, t2[...], preferred_element_type=jnp.float32)
        + b2b[...][None, :], 0.0)
    ob[...] = jnp.mean(h2.reshape(BLK, 32, 32), axis=1)


def _tc1(xpad, t1, b1b, t2, b2b):
    return pl.pallas_call(
        _tc1_body,
        grid=(NT // BLK,),
        in_specs=[
            pl.BlockSpec((BLK, 32), lambda i: (i, 0)),
            pl.BlockSpec((32, 512), lambda i: (0, 0)),
            pl.BlockSpec((512,), lambda i: (0,)),
            pl.BlockSpec((512, 1024), lambda i: (0, 0)),
            pl.BlockSpec((1024,), lambda i: (0,)),
        ],
        out_specs=pl.BlockSpec((BLK, 32), lambda i: (i, 0)),
        out_shape=jax.ShapeDtypeStruct((NT, 32), jnp.float32),
    )(xpad, t1, b1b, t2, b2b)


def _sca_body(h0p, srcp, dstp, acc1, rows_v, sidx, didx, sem, acc):
    cid = lax.axis_index("c")
    sid = lax.axis_index("s")
    wid = cid * 16 + sid
    zeros = jnp.zeros((16,), jnp.float32)

    def zrows(i, carry):
        rows_v[i, pl.ds(0, 16)] = zeros
        rows_v[i, pl.ds(16, 16)] = zeros
        return carry

    lax.fori_loop(0, 512, zrows, 0)

    base = sid * RPT
    for q in range(6):
        pltpu.sync_copy(rows_v, acc.at[pl.ds(base + q * 512, 512)])
    pltpu.sync_copy(rows_v.at[pl.ds(0, RPT - 3072)],
                    acc.at[pl.ds(base + 3072, RPT - 3072)])
    plsc.subcore_barrier()

    def chunk(g, carry):
        e0 = wid * EPW + g * 512
        pltpu.sync_copy(srcp.at[pl.ds(e0, 512)], sidx)
        pltpu.sync_copy(dstp.at[pl.ds(e0, 512)], didx)
        pltpu.async_copy(h0p.at[sidx], rows_v, sem).wait()
        pltpu.sync_copy(rows_v, acc.at[didx], add=True)
        return carry

    lax.fori_loop(0, EPW // 512, chunk, 0)
    plsc.subcore_barrier()

    pltpu.sync_copy(acc.at[pl.ds(base, RPT)], acc1.at[cid, pl.ds(base, RPT)])


def _sca(h0p, srcp, dstp):
    mesh = plsc.VectorSubcoreMesh(core_axis_name="c", subcore_axis_name="s")
    return pl.kernel(
        _sca_body,
        out_type=pltpu.MemorySpace.HBM((2, NT, 32), jnp.float32),
        mesh=mesh,
        compiler_params=_SC_PARAMS,
        scratch_types=[
            pltpu.VMEM((512, 32), jnp.float32),
            pltpu.VMEM((512,), jnp.int32),
            pltpu.VMEM((512,), jnp.int32),
            pltpu.SemaphoreType.DMA,
            pltpu.VMEM_SHARED((NT, 32), jnp.float32),
        ],
    )(h0p, srcp, dstp)


def _scd_body(dstp, degp, hist, didx):
    cid = lax.axis_index("c")
    sid = lax.axis_index("s")
    wid = cid * 16 + sid
    zeros = jnp.zeros((16,), jnp.float32)
    ones = jnp.ones((16,), jnp.float32)

    def zhist(i, carry):
        hist[pl.ds(i * 16, 16)] = zeros
        return carry

    lax.fori_loop(0, NT // 16, zhist, 0)

    def chunk(g, carry):
        e0 = wid * EPW + g * 1024
        pltpu.sync_copy(dstp.at[pl.ds(e0, 1024)], didx)

        def hinc(i, c2):
            dvec = didx[pl.ds(i * 16, 16)]
            plsc.addupdate_scatter(hist, [dvec], ones)
            return c2

        lax.fori_loop(0, 64, hinc, 0)
        return carry

    lax.fori_loop(0, NCHUNK, chunk, 0)
    pltpu.sync_copy(hist, degp.at[wid])


def _scd(dstp):
    mesh = plsc.VectorSubcoreMesh(core_axis_name="c", subcore_axis_name="s")
    return pl.kernel(
        _scd_body,
        out_type=pltpu.MemorySpace.HBM((32, NT), jnp.float32),
        mesh=mesh,
        compiler_params=_SC_PARAMS,
        scratch_types=[
            pltpu.VMEM((NT,), jnp.float32),
            pltpu.VMEM((1024,), jnp.int32),
        ],
    )(dstp)


def _tc2_body(a1b, dgb, h0b, wl1, bl1r, wr1, wlv, wrv, cb, sb, tb, dib):
    a = a1b[0] + a1b[1]
    deg = jnp.sum(dgb[...], axis=0)
    degc = jnp.maximum(deg, 1.0)
    agg = a / degc[:, None]
    h1 = (jnp.einsum("bf,hf->bh", agg, wl1[...],
                     preferred_element_type=jnp.float32)
          + bl1r[...][None, :]
          + jnp.einsum("bf,hf->bh", h0b[...], wr1[...],
                       preferred_element_type=jnp.float32))
    h1 = jnp.maximum(h1, 0.0)
    sb[...] = jnp.sum(h1 * wlv[...][None, :], axis=1)
    tb[...] = jnp.sum(h1 * wrv[...][None, :], axis=1) + cb[0]
    dib[...] = 1.0 / degc


def _tc2(acc1, degp, h0p, Wl1, bl1, Wr1, wl, wr, cc):
    return pl.pallas_call(
        _tc2_body,
        grid=(NT // BLK,),
        in_specs=[
            pl.BlockSpec((2, BLK, 32), lambda i: (0, i, 0)),
            pl.BlockSpec((32, BLK), lambda i: (0, i)),
            pl.BlockSpec((BLK, 32), lambda i: (i, 0)),
            pl.BlockSpec((64, 32), lambda i: (0, 0)),
            pl.BlockSpec((64,), lambda i: (0,)),
            pl.BlockSpec((64, 32), lambda i: (0, 0)),
            pl.BlockSpec((64,), lambda i: (0,)),
            pl.BlockSpec((64,), lambda i: (0,)),
            pl.BlockSpec(memory_space=pltpu.SMEM),
        ],
        out_specs=[
            pl.BlockSpec((BLK,), lambda i: (i,)),
            pl.BlockSpec((BLK,), lambda i: (i,)),
            pl.BlockSpec((BLK,), lambda i: (i,)),
        ],
        out_shape=[
            jax.ShapeDtypeStruct((NT,), jnp.float32),
            jax.ShapeDtypeStruct((NT,), jnp.float32),
            jax.ShapeDtypeStruct((NT,), jnp.float32),
        ],
    )(acc1, degp, h0p, Wl1, bl1, Wr1, wl, wr, cc)


def _scb_body(sp, srcp, dstp, ssump, sv, accv, sidx, didx):
    cid = lax.axis_index("c")
    sid = lax.axis_index("s")
    wid = cid * 16 + sid
    zeros = jnp.zeros((16,), jnp.float32)

    pltpu.sync_copy(sp, sv)

    def zacc(i, carry):
        accv[pl.ds(i * 16, 16)] = zeros
        return carry

    lax.fori_loop(0, NT // 16, zacc, 0)

    def chunk(g, carry):
        e0 = wid * EPW + g * 1024
        pltpu.sync_copy(srcp.at[pl.ds(e0, 1024)], sidx)
        pltpu.sync_copy(dstp.at[pl.ds(e0, 1024)], didx)

        def grp(i, c2):
            svec = plsc.load_gather(sv, [sidx[pl.ds(i * 16, 16)]])
            plsc.addupdate_scatter(accv, [didx[pl.ds(i * 16, 16)]], svec)
            return c2

        lax.fori_loop(0, 64, grp, 0)
        return carry

    lax.fori_loop(0, NCHUNK, chunk, 0)
    pltpu.sync_copy(accv, ssump.at[wid])


def _scb(sp, srcp, dstp):
    mesh = plsc.VectorSubcoreMesh(core_axis_name="c", subcore_axis_name="s")
    return pl.kernel(
        _scb_body,
        out_type=pltpu.MemorySpace.HBM((32, NT), jnp.float32),
        mesh=mesh,
        compiler_params=_SC_PARAMS,
        scratch_types=[
            pltpu.VMEM((NT,), jnp.float32),
            pltpu.VMEM((NT,), jnp.float32),
            pltpu.VMEM((1024,), jnp.int32),
            pltpu.VMEM((1024,), jnp.int32),
        ],
    )(sp, srcp, dstp)


def _tc3_body(ssb, dib, tb, ob):
    ob[...] = jnp.sum(ssb[...], axis=0) * dib[...] + tb[...]


def _tc3(ssump, dinv, t):
    return pl.pallas_call(
        _tc3_body,
        grid=(NT // BLK,),
        in_specs=[
            pl.BlockSpec((32, BLK), lambda i: (0, i)),
            pl.BlockSpec((BLK,), lambda i: (i,)),
            pl.BlockSpec((BLK,), lambda i: (i,)),
        ],
        out_specs=pl.BlockSpec((BLK,), lambda i: (i,)),
        out_shape=jax.ShapeDtypeStruct((NT,), jnp.float32),
    )(ssump, dinv, t)


def kernel(x, edge_index, W1, b1, W2, b2, Wl1, bl1, Wr1, Wl2, bl2, Wr2, Wh, bh):
    src = edge_index[0]
    dst = edge_index[1]
    padv = jnp.full((EP - E,), N, jnp.int32)
    srcp = jnp.concatenate([src, padv])
    dstp = jnp.concatenate([dst, padv])
    xpad = jnp.pad(x, ((0, NT - N), (0, 0)))

    t1 = jnp.zeros((32, 32, 16), jnp.float32)
    t2 = jnp.zeros((32, 16, 32, 32), jnp.float32)
    for k in range(3):
        mk = jnp.eye(32, k=1 - k, dtype=jnp.float32)
        t1 = t1 + mk[:, :, None] * W1[:, 0, k][None, None, :]
        t2 = t2 + mk[:, None, :, None] * W2[:, :, k].T[None, :, None, :]
    t1 = t1.reshape(32, 512)
    t2 = t2.reshape(512, 1024)
    b1b = jnp.tile(b1, 32)
    b2b = jnp.tile(b2, 32)
    wl = (Wh @ Wl2)[0]
    wr = (Wh @ Wr2)[0]
    cc = Wh @ bl2 + bh

    h0p = _tc1(xpad, t1, b1b, t2, b2b)
    degp = _scd(dstp)
    acc1 = _sca(h0p, srcp, dstp)
    s, t, dinv = _tc2(acc1, degp, h0p, Wl1, bl1, Wr1, wl, wr, cc)
    ssump = _scb(s, srcp, dstp)
    outp = _tc3(ssump, dinv, t)
    return outp[:N]


# speedup vs baseline: 16.8340x; 1.0492x over previous
import functools

import jax
import jax.numpy as jnp
from jax import lax
from jax.experimental import pallas as pl
from jax.experimental.pallas import tpu as pltpu
from jax.experimental.pallas import tpu_sc as plsc

N = 50000
E = 1600000
NT = 50176
EP = 1605632
EPW = EP // 32
NCHUNK = EPW // 1024
BLK = 1024
RPT = NT // 16

_SC_PARAMS = pltpu.CompilerParams(
    needs_layout_passes=False, use_tc_tiling_on_sc=False
)


def _tc1_body(xb, t1, b1b, t2, b2b, ob):
    h1 = jnp.maximum(
        jnp.dot(xb[...], t1[...], preferred_element_type=jnp.float32)
        + b1b[...][None, :], 0.0)
    h2 = jnp.maximum(
        jnp.dot(h1, t2[...], preferred_element_type=jnp.float32)
        + b2b[...][None, :], 0.0)
    ob[...] = jnp.mean(h2.reshape(BLK, 32, 32), axis=1)


def _tc1(xpad, t1, b1b, t2, b2b):
    return pl.pallas_call(
        _tc1_body,
        grid=(NT // BLK,),
        in_specs=[
            pl.BlockSpec((BLK, 32), lambda i: (i, 0)),
            pl.BlockSpec((32, 512), lambda i: (0, 0)),
            pl.BlockSpec((512,), lambda i: (0,)),
            pl.BlockSpec((512, 1024), lambda i: (0, 0)),
            pl.BlockSpec((1024,), lambda i: (0,)),
        ],
        out_specs=pl.BlockSpec((BLK, 32), lambda i: (i, 0)),
        out_shape=jax.ShapeDtypeStruct((NT, 32), jnp.float32),
    )(xpad, t1, b1b, t2, b2b)


CHA = 392


def _sca_body(h0p, srcp, dstp, acc1, rows_a, rows_b, sa, da, sb, db,
              sem_ga, sem_gb, sem_sa, sem_sb, acc):
    cid = lax.axis_index("c")
    sid = lax.axis_index("s")
    wid = cid * 16 + sid
    zeros = jnp.zeros((16,), jnp.float32)

    def zrows(i, carry):
        rows_a[i, pl.ds(0, 16)] = zeros
        rows_a[i, pl.ds(16, 16)] = zeros
        return carry

    lax.fori_loop(0, CHA, zrows, 0)

    base = sid * RPT
    for q in range(RPT // CHA):
        pltpu.sync_copy(rows_a, acc.at[pl.ds(base + q * CHA, CHA)])
    plsc.subcore_barrier()

    def chunk2(j, carry):
        e0 = wid * EPW + j * (2 * CHA)
        pltpu.sync_copy(srcp.at[pl.ds(e0, CHA)], sa)
        pltpu.sync_copy(dstp.at[pl.ds(e0, CHA)], da)
        ga = pltpu.async_copy(h0p.at[sa], rows_a, sem_ga)
        pltpu.sync_copy(srcp.at[pl.ds(e0 + CHA, CHA)], sb)
        pltpu.sync_copy(dstp.at[pl.ds(e0 + CHA, CHA)], db)
        gb = pltpu.async_copy(h0p.at[sb], rows_b, sem_gb)
        ga.wait()
        ca = pltpu.async_copy(rows_a, acc.at[da], sem_sa, add=True)
        gb.wait()
        cb = pltpu.async_copy(rows_b, acc.at[db], sem_sb, add=True)
        ca.wait()
        cb.wait()
        return carry

    lax.fori_loop(0, EPW // (2 * CHA), chunk2, 0)
    plsc.subcore_barrier()

    pltpu.sync_copy(acc.at[pl.ds(base, RPT)], acc1.at[cid, pl.ds(base, RPT)])


def _sca(h0p, srcp, dstp):
    mesh = plsc.VectorSubcoreMesh(core_axis_name="c", subcore_axis_name="s")
    return pl.kernel(
        _sca_body,
        out_type=pltpu.MemorySpace.HBM((2, NT, 32), jnp.float32),
        mesh=mesh,
        compiler_params=_SC_PARAMS,
        scratch_types=[
            pltpu.VMEM((CHA, 32), jnp.float32),
            pltpu.VMEM((CHA, 32), jnp.float32),
            pltpu.VMEM((CHA,), jnp.int32),
            pltpu.VMEM((CHA,), jnp.int32),
            pltpu.VMEM((CHA,), jnp.int32),
            pltpu.VMEM((CHA,), jnp.int32),
            pltpu.SemaphoreType.DMA,
            pltpu.SemaphoreType.DMA,
            pltpu.SemaphoreType.DMA,
            pltpu.SemaphoreType.DMA,
            pltpu.VMEM_SHARED((NT, 32), jnp.float32),
        ],
    )(h0p, srcp, dstp)


def _scd_body(dstp, degp, hist, didx):
    cid = lax.axis_index("c")
    sid = lax.axis_index("s")
    wid = cid * 16 + sid
    zeros = jnp.zeros((16,), jnp.float32)
    ones = jnp.ones((16,), jnp.float32)

    def zhist(i, carry):
        hist[pl.ds(i * 16, 16)] = zeros
        return carry

    lax.fori_loop(0, NT // 16, zhist, 0)

    def chunk(g, carry):
        e0 = wid * EPW + g * 1024
        pltpu.sync_copy(dstp.at[pl.ds(e0, 1024)], didx)

        def hinc(i, c2):
            dvec = didx[pl.ds(i * 16, 16)]
            plsc.addupdate_scatter(hist, [dvec], ones)
            return c2

        lax.fori_loop(0, 64, hinc, 0)
        return carry

    lax.fori_loop(0, NCHUNK, chunk, 0)
    pltpu.sync_copy(hist, degp.at[wid])


def _scd(dstp):
    mesh = plsc.VectorSubcoreMesh(core_axis_name="c", subcore_axis_name="s")
    return pl.kernel(
        _scd_body,
        out_type=pltpu.MemorySpace.HBM((32, NT), jnp.float32),
        mesh=mesh,
        compiler_params=_SC_PARAMS,
        scratch_types=[
            pltpu.VMEM((NT,), jnp.float32),
            pltpu.VMEM((1024,), jnp.int32),
        ],
    )(dstp)


def _tc2_body(a1b, dgb, h0b, wl1, bl1r, wr1, wlv, wrv, cb, sb, tb, dib):
    a = a1b[0] + a1b[1]
    deg = jnp.sum(dgb[...], axis=0)
    degc = jnp.maximum(deg, 1.0)
    agg = a / degc[:, None]
    h1 = (jnp.einsum("bf,hf->bh", agg, wl1[...],
                     preferred_element_type=jnp.float32)
          + bl1r[...][None, :]
          + jnp.einsum("bf,hf->bh", h0b[...], wr1[...],
                       preferred_element_type=jnp.float32))
    h1 = jnp.maximum(h1, 0.0)
    sb[...] = jnp.sum(h1 * wlv[...][None, :], axis=1)
    tb[...] = jnp.sum(h1 * wrv[...][None, :], axis=1) + cb[0]
    dib[...] = 1.0 / degc


def _tc2(acc1, degp, h0p, Wl1, bl1, Wr1, wl, wr, cc):
    return pl.pallas_call(
        _tc2_body,
        grid=(NT // BLK,),
        in_specs=[
            pl.BlockSpec((2, BLK, 32), lambda i: (0, i, 0)),
            pl.BlockSpec((32, BLK), lambda i: (0, i)),
            pl.BlockSpec((BLK, 32), lambda i: (i, 0)),
            pl.BlockSpec((64, 32), lambda i: (0, 0)),
            pl.BlockSpec((64,), lambda i: (0,)),
            pl.BlockSpec((64, 32), lambda i: (0, 0)),
            pl.BlockSpec((64,), lambda i: (0,)),
            pl.BlockSpec((64,), lambda i: (0,)),
            pl.BlockSpec(memory_space=pltpu.SMEM),
        ],
        out_specs=[
            pl.BlockSpec((BLK,), lambda i: (i,)),
            pl.BlockSpec((BLK,), lambda i: (i,)),
            pl.BlockSpec((BLK,), lambda i: (i,)),
        ],
        out_shape=[
            jax.ShapeDtypeStruct((NT,), jnp.float32),
            jax.ShapeDtypeStruct((NT,), jnp.float32),
            jax.ShapeDtypeStruct((NT,), jnp.float32),
        ],
    )(acc1, degp, h0p, Wl1, bl1, Wr1, wl, wr, cc)


def _scb_body(sp, srcp, dstp, ssump, sv, accv, sidx, didx):
    cid = lax.axis_index("c")
    sid = lax.axis_index("s")
    wid = cid * 16 + sid
    zeros = jnp.zeros((16,), jnp.float32)

    pltpu.sync_copy(sp, sv)

    def zacc(i, carry):
        accv[pl.ds(i * 16, 16)] = zeros
        return carry

    lax.fori_loop(0, NT // 16, zacc, 0)

    def chunk(g, carry):
        e0 = wid * EPW + g * 1024
        pltpu.sync_copy(srcp.at[pl.ds(e0, 1024)], sidx)
        pltpu.sync_copy(dstp.at[pl.ds(e0, 1024)], didx)

        def grp(i, c2):
            svec = plsc.load_gather(sv, [sidx[pl.ds(i * 16, 16)]])
            plsc.addupdate_scatter(accv, [didx[pl.ds(i * 16, 16)]], svec)
            return c2

        lax.fori_loop(0, 64, grp, 0)
        return carry

    lax.fori_loop(0, NCHUNK, chunk, 0)
    pltpu.sync_copy(accv, ssump.at[wid])


def _scb(sp, srcp, dstp):
    mesh = plsc.VectorSubcoreMesh(core_axis_name="c", subcore_axis_name="s")
    return pl.kernel(
        _scb_body,
        out_type=pltpu.MemorySpace.HBM((32, NT), jnp.float32),
        mesh=mesh,
        compiler_params=_SC_PARAMS,
        scratch_types=[
            pltpu.VMEM((NT,), jnp.float32),
            pltpu.VMEM((NT,), jnp.float32),
            pltpu.VMEM((1024,), jnp.int32),
            pltpu.VMEM((1024,), jnp.int32),
        ],
    )(sp, srcp, dstp)


def _tc3_body(ssb, dib, tb, ob):
    ob[...] = jnp.sum(ssb[...], axis=0) * dib[...] + tb[...]


def _tc3(ssump, dinv, t):
    return pl.pallas_call(
        _tc3_body,
        grid=(NT // BLK,),
        in_specs=[
            pl.BlockSpec((32, BLK), lambda i: (0, i)),
            pl.BlockSpec((BLK,), lambda i: (i,)),
            pl.BlockSpec((BLK,), lambda i: (i,)),
        ],
        out_specs=pl.BlockSpec((BLK,), lambda i: (i,)),
        out_shape=jax.ShapeDtypeStruct((NT,), jnp.float32),
    )(ssump, dinv, t)


def kernel(x, edge_index, W1, b1, W2, b2, Wl1, bl1, Wr1, Wl2, bl2, Wr2, Wh, bh):
    src = edge_index[0]
    dst = edge_index[1]
    padv = jnp.full((EP - E,), N, jnp.int32)
    srcp = jnp.concatenate([src, padv])
    dstp = jnp.concatenate([dst, padv])
    xpad = jnp.pad(x, ((0, NT - N), (0, 0)))

    t1 = jnp.zeros((32, 32, 16), jnp.float32)
    t2 = jnp.zeros((32, 16, 32, 32), jnp.float32)
    for k in range(3):
        mk = jnp.eye(32, k=1 - k, dtype=jnp.float32)
        t1 = t1 + mk[:, :, None] * W1[:, 0, k][None, None, :]
        t2 = t2 + mk[:, None, :, None] * W2[:, :, k].T[None, :, None, :]
    t1 = t1.reshape(32, 512)
    t2 = t2.reshape(512, 1024)
    b1b = jnp.tile(b1, 32)
    b2b = jnp.tile(b2, 32)
    wl = (Wh @ Wl2)[0]
    wr = (Wh @ Wr2)[0]
    cc = Wh @ bl2 + bh

    h0p = _tc1(xpad, t1, b1b, t2, b2b)
    degp = _scd(dstp)
    acc1 = _sca(h0p, srcp, dstp)
    s, t, dinv = _tc2(acc1, degp, h0p, Wl1, bl1, Wr1, wl, wr, cc)
    ssump = _scb(s, srcp, dstp)
    outp = _tc3(ssump, dinv, t)
    return outp[:N]
